# baseline (speedup 1.0000x reference)
import functools

import jax
import jax.numpy as jnp
from jax.experimental import pallas as pl
from jax.experimental.pallas import tpu as pltpu

IN_DIM = 1024
OUT_DIM = 1024
E = 8
TOP_K = 2


def _router_body(x_ref, gw1_ref, gb1_ref, gw2_ref, gb2_ref, ew_ref,
                 logits_ref, cw_ref, ewb_ref):
    x = x_ref[...]
    hidden = jnp.dot(x, gw1_ref[...], preferred_element_type=jnp.float32)
    hidden = hidden + gb1_ref[...]
    logits = jnp.dot(hidden, gw2_ref[...], preferred_element_type=jnp.float32)
    logits = logits + gb2_ref[...]
    logits_ref[...] = logits

    m = jnp.max(logits, axis=-1, keepdims=True)
    ex = jnp.exp(logits - m)
    probs = ex / jnp.sum(ex, axis=-1, keepdims=True)

    e_iota = jax.lax.broadcasted_iota(jnp.int32, probs.shape, 1)
    m1 = jnp.max(probs, axis=-1, keepdims=True)
    is1 = (probs == m1)
    a1 = jnp.min(jnp.where(is1, e_iota, E), axis=-1, keepdims=True)
    masked = jnp.where(e_iota == a1, -jnp.inf, probs)
    m2 = jnp.max(masked, axis=-1, keepdims=True)
    is2 = (masked == m2)
    a2 = jnp.min(jnp.where(is2, e_iota, E), axis=-1, keepdims=True)
    denom = m1 + m2
    w1 = m1 / denom
    w2 = m2 / denom
    cw_ref[...] = (jnp.where(e_iota == a1, w1, 0.0)
                   + jnp.where(e_iota == a2, w2, 0.0))
    ewb_ref[...] = ew_ref[...].astype(jnp.bfloat16)


def _dispatch_body(x_ref, cw_ref, ewb_ref, eb_ref, out_ref):
    x = x_ref[...]
    cw = cw_ref[...]
    xb = x.astype(jnp.bfloat16)
    acc = jnp.zeros((x.shape[0], OUT_DIM), dtype=jnp.float32)
    for e in range(E):
        eo = jnp.dot(xb, ewb_ref[e], preferred_element_type=jnp.float32)
        eo = eo + eb_ref[e:e + 1, :]
        acc = acc + eo * cw[:, e:e + 1]
    out_ref[...] = acc


@functools.partial(jax.jit, static_argnames=())
def kernel(x, gate_w1, gate_b1, gate_w2, gate_b2, expert_w, expert_b):
    b, s, h = x.shape
    n = b * s
    flat = x.reshape(n, h)
    tb_a = n // E
    logits, cw, ewb = pl.pallas_call(
        _router_body,
        grid=(E,),
        in_specs=[
            pl.BlockSpec((tb_a, h), lambda i: (i, 0)),
            pl.BlockSpec((h, h // 2), lambda i: (0, 0)),
            pl.BlockSpec((1, h // 2), lambda i: (0, 0)),
            pl.BlockSpec((h // 2, E), lambda i: (0, 0)),
            pl.BlockSpec((1, E), lambda i: (0, 0)),
            pl.BlockSpec((1, h, OUT_DIM), lambda i: (i, 0, 0)),
        ],
        out_specs=(
            pl.BlockSpec((tb_a, E), lambda i: (i, 0)),
            pl.BlockSpec((tb_a, E), lambda i: (i, 0)),
            pl.BlockSpec((1, h, OUT_DIM), lambda i: (i, 0, 0)),
        ),
        out_shape=(
            jax.ShapeDtypeStruct((n, E), jnp.float32),
            jax.ShapeDtypeStruct((n, E), jnp.float32),
            jax.ShapeDtypeStruct((E, h, OUT_DIM), jnp.bfloat16),
        ),
        compiler_params=pltpu.CompilerParams(
            dimension_semantics=("arbitrary",),
        ),
    )(flat, gate_w1, gate_b1.reshape(1, -1), gate_w2, gate_b2.reshape(1, -1),
      expert_w)

    tb_b = 256
    final = pl.pallas_call(
        _dispatch_body,
        grid=(n // tb_b,),
        in_specs=[
            pl.BlockSpec((tb_b, h), lambda i: (i, 0)),
            pl.BlockSpec((tb_b, E), lambda i: (i, 0)),
            pl.BlockSpec((E, h, OUT_DIM), lambda i: (0, 0, 0)),
            pl.BlockSpec((E, OUT_DIM), lambda i: (0, 0)),
        ],
        out_specs=pl.BlockSpec((tb_b, OUT_DIM), lambda i: (i, 0)),
        out_shape=jax.ShapeDtypeStruct((n, OUT_DIM), jnp.float32),
        compiler_params=pltpu.CompilerParams(
            dimension_semantics=("arbitrary",),
        ),
    )(flat, cw, ewb, expert_b)
    return final.reshape(b, s, OUT_DIM), logits


# speedup vs baseline: 1.1016x; 1.1016x over previous
import functools

import jax
import jax.numpy as jnp
from jax.experimental import pallas as pl
from jax.experimental.pallas import tpu as pltpu

IN_DIM = 1024
OUT_DIM = 1024
E = 8
TOP_K = 2
CHUNK = 512


def _moe_body(x_hbm, gw1_ref, gb1_ref, gw2_ref, gb2_ref, ew_ref, eb_ref,
              out_ref, logits_ref, xb_ref, cw_ref, xc_refs, sems):
    e = pl.program_id(0)
    n = out_ref.shape[0]
    nch = n // CHUNK

    @pl.when(e == 0)
    def _router():
        copies = [
            pltpu.make_async_copy(
                x_hbm.at[pl.ds(i * CHUNK, CHUNK), :], xc_refs[i % 2],
                sems[i % 2])
            for i in range(nch)
        ]
        copies[0].start()
        for i in range(nch):
            copies[i].wait()
            xc = xc_refs[i % 2][...]
            if i + 1 < nch:
                copies[i + 1].start()
            hidden = jnp.dot(xc, gw1_ref[...],
                             preferred_element_type=jnp.float32)
            logits = jnp.dot(hidden + gb1_ref[...], gw2_ref[...],
                             preferred_element_type=jnp.float32)
            logits = logits + gb2_ref[...]
            logits_ref[pl.ds(i * CHUNK, CHUNK), :] = logits

            m = jnp.max(logits, axis=-1, keepdims=True)
            exl = jnp.exp(logits - m)
            probs = exl / jnp.sum(exl, axis=-1, keepdims=True)
            e_iota = jax.lax.broadcasted_iota(jnp.int32, probs.shape, 1)
            m1 = jnp.max(probs, axis=-1, keepdims=True)
            a1 = jnp.min(jnp.where(probs == m1, e_iota, E), axis=-1,
                         keepdims=True)
            masked = jnp.where(e_iota == a1, -jnp.inf, probs)
            m2 = jnp.max(masked, axis=-1, keepdims=True)
            a2 = jnp.min(jnp.where(masked == m2, e_iota, E), axis=-1,
                         keepdims=True)
            denom = m1 + m2
            cw_ref[pl.ds(i * CHUNK, CHUNK), :] = (
                jnp.where(e_iota == a1, m1 / denom, 0.0)
                + jnp.where(e_iota == a2, m2 / denom, 0.0))
            xb_ref[pl.ds(i * CHUNK, CHUNK), :] = xc.astype(jnp.bfloat16)

    def _contrib(i):
        rows = pl.ds(i * CHUNK, CHUNK)
        eo = jnp.dot(xb_ref[rows, :], ew_ref[0],
                     preferred_element_type=jnp.float32)
        eo = eo + eb_ref[0]
        cwc = cw_ref[rows, :]
        col_iota = jax.lax.broadcasted_iota(jnp.int32, cwc.shape, 1)
        cw_col = jnp.sum(jnp.where(col_iota == e, cwc, 0.0), axis=1,
                         keepdims=True)
        return rows, eo * cw_col

    @pl.when(e == 0)
    def _init():
        for i in range(nch):
            rows, contrib = _contrib(i)
            out_ref[rows, :] = contrib

    @pl.when(e != 0)
    def _acc():
        for i in range(nch):
            rows, contrib = _contrib(i)
            out_ref[rows, :] = out_ref[rows, :] + contrib


@functools.partial(jax.jit, static_argnames=())
def kernel(x, gate_w1, gate_b1, gate_w2, gate_b2, expert_w, expert_b):
    b, s, h = x.shape
    n = b * s
    flat = x.reshape(n, h)

    def body(x_hbm, gw1, gb1, gw2, gb2, ew, eb, out, logits,
             xb, cw, xc0, xc1, sem0, sem1):
        _moe_body(x_hbm, gw1, gb1, gw2, gb2, ew, eb, out, logits,
                  xb, cw, (xc0, xc1), (sem0, sem1))

    final, logits = pl.pallas_call(
        body,
        grid=(E,),
        in_specs=[
            pl.BlockSpec(memory_space=pl.ANY),
            pl.BlockSpec((h, h // 2), lambda e: (0, 0)),
            pl.BlockSpec((1, h // 2), lambda e: (0, 0)),
            pl.BlockSpec((h // 2, E), lambda e: (0, 0)),
            pl.BlockSpec((1, E), lambda e: (0, 0)),
            pl.BlockSpec((1, h, OUT_DIM), lambda e: (e, 0, 0)),
            pl.BlockSpec((1, 1, OUT_DIM), lambda e: (e, 0, 0)),
        ],
        out_specs=(
            pl.BlockSpec((n, OUT_DIM), lambda e: (0, 0)),
            pl.BlockSpec((n, E), lambda e: (0, 0)),
        ),
        out_shape=(
            jax.ShapeDtypeStruct((n, OUT_DIM), jnp.float32),
            jax.ShapeDtypeStruct((n, E), jnp.float32),
        ),
        scratch_shapes=[
            pltpu.VMEM((n, h), jnp.bfloat16),
            pltpu.VMEM((n, E), jnp.float32),
            pltpu.VMEM((CHUNK, h), jnp.float32),
            pltpu.VMEM((CHUNK, h), jnp.float32),
            pltpu.SemaphoreType.DMA,
            pltpu.SemaphoreType.DMA,
        ],
        compiler_params=pltpu.CompilerParams(
            dimension_semantics=("arbitrary",),
        ),
    )(flat, gate_w1, gate_b1.reshape(1, -1), gate_w2, gate_b2.reshape(1, -1),
      expert_w, expert_b.reshape(E, 1, OUT_DIM))
    return final.reshape(b, s, OUT_DIM), logits
